# Initial kernel scaffold; baseline (speedup 1.0000x reference)
import functools

import numpy as np
import jax
import jax.numpy as jnp
from jax import lax
from jax.experimental import pallas as pl
from jax.experimental.pallas import tpu as pltpu
from jax.experimental.pallas import tpu_sc as plsc

B = 8
N = 96 * 224 * 224
NC = 2
NS = 16
NW = NC * NS
TPS = NW // B
M = N // TPS
BLK = 4096
NBLK = M // BLK
LANES = 16
NSLOT = 17
DEAD = 16
HSZ = NSLOT * 256 * LANES

_QL = np.array([0.0, 0.01, 0.1, 0.25, 0.5, 0.75, 0.9, 0.99, 1.0], np.float32)
_pos = (_QL * np.float32(N - 1)).astype(np.float32)
_lo = np.floor(_pos).astype(np.int64)
_hi = np.ceil(_pos).astype(np.int64)
RANKS = sorted(set(_lo.tolist()) | set(_hi.tolist()))
assert len(RANKS) == 16, RANKS
_RIDX = {r: i for i, r in enumerate(RANKS)}
LO_IDX = np.array([_RIDX[int(r)] for r in _lo], np.int64)
HI_IDX = np.array([_RIDX[int(r)] for r in _hi], np.int64)
FRAC = (_pos - np.floor(_pos)).astype(np.float32)
RANKS_ARR = np.array(RANKS, np.int32)


def _make_pass(nlut):
    slots = 1 if nlut == 0 else NSLOT
    hsz = slots * 256 * LANES
    lut_words = []
    if nlut >= 1:
        lut_words.append(256)
    for _ in range(1, nlut):
        lut_words.append(NSLOT * 256)

    scratch = [pltpu.VMEM((BLK,), jnp.float32)]
    scratch += [pltpu.VMEM((w,), jnp.int32) for w in lut_words]
    scratch += [pltpu.VMEM((hsz,), jnp.int32)]

    mesh = plsc.VectorSubcoreMesh(core_axis_name="c", subcore_axis_name="s")

    @functools.partial(
        pl.kernel,
        out_type=jax.ShapeDtypeStruct((NW, hsz), jnp.int32),
        mesh=mesh,
        scratch_types=scratch,
    )
    def body(x_hbm, *rest):
        lut_hbms = rest[:nlut]
        out_hbm = rest[nlut]
        data_v = rest[nlut + 1]
        lut_vs = rest[nlut + 2:nlut + 2 + nlut]
        hist_v = rest[nlut + 2 + nlut]

        cid = lax.axis_index("c")
        sid = lax.axis_index("s")
        wid = sid * NC + cid
        sample = wid // TPS
        base = wid * M

        zero16 = jnp.zeros((16,), jnp.int32)

        def _zero(j, c):
            hist_v[pl.ds(j * 16, 16)] = zero16
            return c

        lax.fori_loop(0, hsz // 16, _zero, 0)

        for lv, lh, w in zip(lut_vs, lut_hbms, lut_words):
            pltpu.sync_copy(lh.at[pl.ds(sample * w, w)], lv)

        lane = lax.iota(jnp.int32, 16)
        one16 = jnp.ones((16,), jnp.int32)

        def _blk(i, c):
            pltpu.sync_copy(x_hbm.at[pl.ds(base + i * BLK, BLK)], data_v)

            def _vec(j, c2):
                v = data_v[pl.ds(j * 16, 16)]
                bits = plsc.bitcast(v, jnp.uint32)
                neg = bits >> 31
                key = bits ^ ((jnp.uint32(0) - neg) | jnp.uint32(0x80000000))
                if nlut == 0:
                    binr = plsc.bitcast(key >> 24, jnp.int32)
                    idx = binr * 16 + lane
                else:
                    b0 = plsc.bitcast(key >> 24, jnp.int32)
                    slot = plsc.load_gather(lut_vs[0], [b0])
                    shift = 16
                    for l in range(1, nlut):
                        bl = plsc.bitcast((key >> shift) & jnp.uint32(255),
                                          jnp.int32)
                        slot = plsc.load_gather(lut_vs[l], [slot * 256 + bl])
                        shift -= 8
                    binr = plsc.bitcast((key >> shift) & jnp.uint32(255),
                                        jnp.int32)
                    idx = (slot * 256 + binr) * 16 + lane
                plsc.addupdate_scatter(hist_v, [idx], one16)
                return c2

            lax.fori_loop(0, BLK // 16, _vec, 0)
            return c

        lax.fori_loop(0, NBLK, _blk, 0)

        pltpu.sync_copy(hist_v, out_hbm.at[wid])

    return body


_pass0 = _make_pass(0)
_pass1 = _make_pass(1)
_pass2 = _make_pass(2)
_pass3 = _make_pass(3)


def _digit_step(rows, resid):
    cum = jnp.cumsum(rows, axis=-1)
    digit = jnp.sum((cum <= resid[..., None]).astype(jnp.int32), axis=-1)
    excl = cum - rows
    below = jnp.take_along_axis(excl, digit[..., None], axis=-1)[..., 0]
    return digit, resid - below


def _first_occurrence(keys):
    return jnp.sum((keys[:, None, :] < keys[:, :, None]).astype(jnp.int32),
                   axis=-1)


def _reduce_hist(ht, slots):
    return ht.reshape(B, TPS, slots, 256, LANES).sum(axis=(1, 4))


def kernel(x):
    xf = x.reshape(-1)
    barr = jnp.arange(B)[:, None]
    ranks = jnp.broadcast_to(jnp.asarray(RANKS_ARR)[None, :], (B, 16))

    h0 = _reduce_hist(_pass0(xf), 1)[:, 0]
    rows0 = jnp.broadcast_to(h0[:, None, :], (B, 16, 256))
    b1, r1 = _digit_step(rows0, ranks)
    eff1 = _first_occurrence(b1)
    lutA = jnp.full((B, 256), DEAD, jnp.int32).at[barr, b1].set(eff1)

    h1 = _reduce_hist(_pass1(xf, lutA.reshape(-1)), NSLOT)
    rows1 = h1[barr, eff1]
    b2, r2 = _digit_step(rows1, r1)
    key2 = eff1 * 256 + b2
    eff2 = _first_occurrence(key2)
    lutB = jnp.full((B, NSLOT * 256), DEAD, jnp.int32).at[barr, key2].set(eff2)

    h2 = _reduce_hist(_pass2(xf, lutA.reshape(-1), lutB.reshape(-1)), NSLOT)
    rows2 = h2[barr, eff2]
    b3, r3 = _digit_step(rows2, r2)
    key3 = eff2 * 256 + b3
    eff3 = _first_occurrence(key3)
    lutC = jnp.full((B, NSLOT * 256), DEAD, jnp.int32).at[barr, key3].set(eff3)

    h3 = _reduce_hist(
        _pass3(xf, lutA.reshape(-1), lutB.reshape(-1), lutC.reshape(-1)),
        NSLOT)
    rows3 = h3[barr, eff3]
    b4, _ = _digit_step(rows3, r3)

    key32 = ((b1.astype(jnp.uint32) << 24) | (b2.astype(jnp.uint32) << 16)
             | (b3.astype(jnp.uint32) << 8) | b4.astype(jnp.uint32))
    pos_mask = key32 >> 31
    flip = jnp.where(pos_mask == 1, jnp.uint32(0x80000000),
                     jnp.uint32(0xFFFFFFFF))
    vals = lax.bitcast_convert_type(key32 ^ flip, jnp.float32)

    vlo = vals[:, jnp.asarray(LO_IDX)]
    vhi = vals[:, jnp.asarray(HI_IDX)]
    frac = jnp.asarray(FRAC)[None, :]
    return vlo * (jnp.float32(1) - frac) + vhi * frac


# speedup vs baseline: 9.0761x; 9.0761x over previous
import functools

import numpy as np
import jax
import jax.numpy as jnp
from jax import lax
from jax.experimental import pallas as pl
from jax.experimental.pallas import tpu as pltpu
from jax.experimental.pallas import tpu_sc as plsc

B = 8
N = 96 * 224 * 224
NC = 2
NS = 16
NW = NC * NS
TPS = NW // B
M = N // TPS
BLK = 4096
NBLK = M // BLK
LANES = 16
NSLOT = 17
DEAD = 16
HSZ = NSLOT * 256 * LANES

_QL = np.array([0.0, 0.01, 0.1, 0.25, 0.5, 0.75, 0.9, 0.99, 1.0], np.float32)
_pos = (_QL * np.float32(N - 1)).astype(np.float32)
_lo = np.floor(_pos).astype(np.int64)
_hi = np.ceil(_pos).astype(np.int64)
RANKS = sorted(set(_lo.tolist()) | set(_hi.tolist()))
while len(RANKS) < 16:
    RANKS.append(RANKS[-1])
assert len(RANKS) == 16, RANKS
_RIDX = {r: i for i, r in enumerate(RANKS)}
LO_IDX = np.array([_RIDX[int(r)] for r in _lo], np.int64)
HI_IDX = np.array([_RIDX[int(r)] for r in _hi], np.int64)
FRAC = (_pos - np.floor(_pos)).astype(np.float32)
RANKS_ARR = np.array(RANKS, np.int32)


@functools.lru_cache(maxsize=None)
def _make_pass(nlut):
    slots = 1 if nlut == 0 else NSLOT
    hsz = slots * 256 * LANES
    lut_words = []
    if nlut >= 1:
        lut_words.append(256)
    for _ in range(1, nlut):
        lut_words.append(NSLOT * 256)

    scratch = [pltpu.VMEM((BLK,), jnp.int32)]
    scratch += [pltpu.VMEM((w,), jnp.int32) for w in lut_words]
    scratch += [pltpu.VMEM((hsz,), jnp.int32)]

    mesh = plsc.VectorSubcoreMesh(core_axis_name="c", subcore_axis_name="s")

    @functools.partial(
        pl.kernel,
        out_type=jax.ShapeDtypeStruct((NW, hsz), jnp.int32),
        mesh=mesh,
        scratch_types=scratch,
        compiler_params=pltpu.CompilerParams(needs_layout_passes=False),
    )
    def body(x_hbm, *rest):
        lut_hbms = rest[:nlut]
        out_hbm = rest[nlut]
        data_v = rest[nlut + 1]
        lut_vs = rest[nlut + 2:nlut + 2 + nlut]
        hist_v = rest[nlut + 2 + nlut]

        cid = lax.axis_index("c")
        sid = lax.axis_index("s")
        wid = sid * NC + cid
        sample = wid // TPS
        base = wid * M

        zero16 = jnp.zeros((16,), jnp.int32)

        def _zero(j, c):
            hist_v[pl.ds(j * 16, 16)] = zero16
            return c

        lax.fori_loop(0, hsz // 16, _zero, 0)

        for lv, lh, w in zip(lut_vs, lut_hbms, lut_words):
            pltpu.sync_copy(lh.at[pl.ds(sample * w, w)], lv)

        lane = lax.iota(jnp.int32, 16)
        one16 = jnp.ones((16,), jnp.int32)
        c31 = jnp.full((16,), 31, jnp.int32)
        cmin = jnp.full((16,), -2147483648, jnp.int32)
        c255 = jnp.full((16,), 255, jnp.int32)

        def _digit(key, sh):
            d = lax.shift_right_logical(key, jnp.full((16,), sh, jnp.int32))
            return lax.bitwise_and(d, c255) if sh < 24 else d

        def _blk(i, c):
            pltpu.sync_copy(x_hbm.at[pl.ds(base + i * BLK, BLK)], data_v)

            def _vec(j, c2):
                bits = data_v[pl.ds(j * 16, 16)]
                m = lax.shift_right_arithmetic(bits, c31)
                key = lax.bitwise_xor(bits, lax.bitwise_or(m, cmin))
                if nlut == 0:
                    idx = _digit(key, 24) * 16 + lane
                else:
                    slot = plsc.load_gather(lut_vs[0], [_digit(key, 24)])
                    shift = 16
                    for l in range(1, nlut):
                        slot = plsc.load_gather(
                            lut_vs[l], [slot * 256 + _digit(key, shift)])
                        shift -= 8
                    idx = (slot * 256 + _digit(key, shift)) * 16 + lane
                plsc.addupdate_scatter(hist_v, [idx], one16)
                return c2

            lax.fori_loop(0, BLK // 16, _vec, 0)
            return c

        lax.fori_loop(0, NBLK, _blk, 0)

        pltpu.sync_copy(hist_v, out_hbm.at[wid])

    return body


def _digit_step(rows, resid):
    cum = jnp.cumsum(rows, axis=-1)
    digit = jnp.sum((cum <= resid[..., None]).astype(jnp.int32), axis=-1)
    excl = cum - rows
    below = jnp.take_along_axis(excl, digit[..., None], axis=-1)[..., 0]
    return digit, resid - below


def _first_occurrence(keys):
    return jnp.sum((keys[:, None, :] < keys[:, :, None]).astype(jnp.int32),
                   axis=-1)


def _reduce_hist(ht, slots):
    return ht.reshape(B, TPS, slots, 256, LANES).sum(axis=(1, 4))


def kernel(x):
    xf = lax.bitcast_convert_type(x, jnp.int32).reshape(-1)
    barr = jnp.arange(B)[:, None]
    ranks = jnp.broadcast_to(jnp.asarray(RANKS_ARR)[None, :], (B, 16))

    h0 = _reduce_hist(_make_pass(0)(xf), 1)[:, 0]
    rows0 = jnp.broadcast_to(h0[:, None, :], (B, 16, 256))
    b1, r1 = _digit_step(rows0, ranks)
    eff1 = _first_occurrence(b1)
    lutA = jnp.full((B, 256), DEAD, jnp.int32).at[barr, b1].set(eff1)

    h1 = _reduce_hist(_make_pass(1)(xf, lutA.reshape(-1)), NSLOT)
    rows1 = h1[barr, eff1]
    b2, r2 = _digit_step(rows1, r1)
    key2 = eff1 * 256 + b2
    eff2 = _first_occurrence(key2)
    lutB = jnp.full((B, NSLOT * 256), DEAD, jnp.int32).at[barr, key2].set(eff2)

    h2 = _reduce_hist(_make_pass(2)(xf, lutA.reshape(-1), lutB.reshape(-1)), NSLOT)
    rows2 = h2[barr, eff2]
    b3, r3 = _digit_step(rows2, r2)
    key3 = eff2 * 256 + b3
    eff3 = _first_occurrence(key3)
    lutC = jnp.full((B, NSLOT * 256), DEAD, jnp.int32).at[barr, key3].set(eff3)

    h3 = _reduce_hist(
        _make_pass(3)(xf, lutA.reshape(-1), lutB.reshape(-1), lutC.reshape(-1)),
        NSLOT)
    rows3 = h3[barr, eff3]
    b4, _ = _digit_step(rows3, r3)

    key32 = ((b1.astype(jnp.uint32) << 24) | (b2.astype(jnp.uint32) << 16)
             | (b3.astype(jnp.uint32) << 8) | b4.astype(jnp.uint32))
    pos_mask = key32 >> 31
    flip = jnp.where(pos_mask == 1, jnp.uint32(0x80000000),
                     jnp.uint32(0xFFFFFFFF))
    vals = lax.bitcast_convert_type(key32 ^ flip, jnp.float32)

    vlo = vals[:, jnp.asarray(LO_IDX)]
    vhi = vals[:, jnp.asarray(HI_IDX)]
    frac = jnp.asarray(FRAC)[None, :]
    return vlo * (jnp.float32(1) - frac) + vhi * frac


# speedup vs baseline: 18.1747x; 2.0025x over previous
import functools

import numpy as np
import jax
import jax.numpy as jnp
from jax import lax
from jax.experimental import pallas as pl
from jax.experimental.pallas import tpu as pltpu
from jax.experimental.pallas import tpu_sc as plsc

B = 8
N = 96 * 224 * 224
NC = 2
NS = 16
NW = NC * NS
TPS = NW // B
M = N // TPS
BLK = 4096
NBLK = M // BLK
NSLOT = 17
DEAD = 16
D0 = 4096
D1 = 4096
D2 = 256
UNROLL = 8

_QL = np.array([0.0, 0.01, 0.1, 0.25, 0.5, 0.75, 0.9, 0.99, 1.0], np.float32)
_pos = (_QL * np.float32(N - 1)).astype(np.float32)
_lo = np.floor(_pos).astype(np.int64)
_hi = np.ceil(_pos).astype(np.int64)
RANKS = sorted(set(_lo.tolist()) | set(_hi.tolist()))
while len(RANKS) < 16:
    RANKS.append(RANKS[-1])
assert len(RANKS) == 16, RANKS
_RIDX = {r: i for i, r in enumerate(RANKS)}
LO_IDX = np.array([_RIDX[int(r)] for r in _lo], np.int64)
HI_IDX = np.array([_RIDX[int(r)] for r in _hi], np.int64)
FRAC = (_pos - np.floor(_pos)).astype(np.float32)
RANKS_ARR = np.array(RANKS, np.int32)


@functools.lru_cache(maxsize=None)
def _make_pass(p):
    hsz = (D0, NSLOT * D1, NSLOT * D2)[p]
    lut_words = ((), (D0,), (D0, NSLOT * D1))[p]

    scratch = [pltpu.VMEM((BLK,), jnp.int32), pltpu.VMEM((BLK,), jnp.int32)]
    scratch += [pltpu.VMEM((w,), jnp.int32) for w in lut_words]
    scratch += [pltpu.VMEM((hsz,), jnp.int32),
                pltpu.SemaphoreType.DMA, pltpu.SemaphoreType.DMA]

    mesh = plsc.VectorSubcoreMesh(core_axis_name="c", subcore_axis_name="s")
    nlut = len(lut_words)

    @functools.partial(
        pl.kernel,
        out_type=jax.ShapeDtypeStruct((NW, hsz), jnp.int32),
        mesh=mesh,
        scratch_types=scratch,
        compiler_params=pltpu.CompilerParams(needs_layout_passes=False),
    )
    def body(x_hbm, *rest):
        lut_hbms = rest[:nlut]
        out_hbm = rest[nlut]
        buf0_v = rest[nlut + 1]
        buf1_v = rest[nlut + 2]
        lut_vs = rest[nlut + 3:nlut + 3 + nlut]
        hist_v, sem0, sem1 = rest[nlut + 3 + nlut:]

        cid = lax.axis_index("c")
        sid = lax.axis_index("s")
        wid = sid * NC + cid
        sample = wid // TPS
        base = wid * M

        pltpu.async_copy(x_hbm.at[pl.ds(base, BLK)], buf0_v, sem0)

        zero16 = jnp.zeros((16,), jnp.int32)

        def _zero(j, c):
            for u in range(UNROLL):
                hist_v[pl.ds((j * UNROLL + u) * 16, 16)] = zero16
            return c

        lax.fori_loop(0, hsz // 16 // UNROLL, _zero, 0)

        for lv, lh, w in zip(lut_vs, lut_hbms, lut_words):
            pltpu.sync_copy(lh.at[pl.ds(sample * w, w)], lv)

        one16 = jnp.ones((16,), jnp.int32)
        c31 = jnp.full((16,), 31, jnp.int32)
        cmin = jnp.full((16,), -2147483648, jnp.int32)

        def _digits(key):
            top = lax.shift_right_logical(key, jnp.full((16,), 20, jnp.int32))
            if p == 0:
                return (), top
            mid = lax.bitwise_and(
                lax.shift_right_logical(key, jnp.full((16,), 8, jnp.int32)),
                jnp.full((16,), 4095, jnp.int32))
            if p == 1:
                return (top,), mid
            low = lax.bitwise_and(key, jnp.full((16,), 255, jnp.int32))
            return (top, mid), low

        def _process(buf):
            def _vec(j, c2):
                for u in range(UNROLL):
                    bits = buf[pl.ds((j * UNROLL + u) * 16, 16)]
                    m = lax.shift_right_arithmetic(bits, c31)
                    key = lax.bitwise_xor(bits, lax.bitwise_or(m, cmin))
                    prefix, digit = _digits(key)
                    if p == 0:
                        idx = digit
                    else:
                        slot = plsc.load_gather(lut_vs[0], [prefix[0]])
                        if p == 2:
                            slot = plsc.load_gather(
                                lut_vs[1], [slot * D1 + prefix[1]])
                        idx = slot * (D1 if p == 1 else D2) + digit
                    plsc.addupdate_scatter(hist_v, [idx], one16)
                return c2

            lax.fori_loop(0, BLK // 16 // UNROLL, _vec, 0)

        def _pair(q, c):
            i = q * 2
            pltpu.async_copy(x_hbm.at[pl.ds(base + (i + 1) * BLK, BLK)],
                             buf1_v, sem1)
            pltpu.make_async_copy(x_hbm.at[pl.ds(base + i * BLK, BLK)],
                                  buf0_v, sem0).wait()
            _process(buf0_v)
            nxt = jnp.where(i + 2 >= NBLK, 0, i + 2)
            pltpu.async_copy(x_hbm.at[pl.ds(base + nxt * BLK, BLK)],
                             buf0_v, sem0)
            pltpu.make_async_copy(x_hbm.at[pl.ds(base + (i + 1) * BLK, BLK)],
                                  buf1_v, sem1).wait()
            _process(buf1_v)
            return c

        lax.fori_loop(0, NBLK // 2, _pair, 0)

        pltpu.make_async_copy(x_hbm.at[pl.ds(base, BLK)], buf0_v,
                              sem0).wait()

        pltpu.sync_copy(hist_v, out_hbm.at[wid])

    return body


def _digit_step(rows, resid):
    cum = jnp.cumsum(rows, axis=-1)
    digit = jnp.sum((cum <= resid[..., None]).astype(jnp.int32), axis=-1)
    excl = cum - rows
    below = jnp.take_along_axis(excl, digit[..., None], axis=-1)[..., 0]
    return digit, resid - below


def _first_occurrence(keys):
    return jnp.sum((keys[:, None, :] < keys[:, :, None]).astype(jnp.int32),
                   axis=-1)


def kernel(x):
    xf = lax.bitcast_convert_type(x, jnp.int32).reshape(-1)
    barr = jnp.arange(B)[:, None]
    ranks = jnp.broadcast_to(jnp.asarray(RANKS_ARR)[None, :], (B, 16))

    h0 = _make_pass(0)(xf).reshape(B, TPS, D0).sum(axis=1)
    rows0 = jnp.broadcast_to(h0[:, None, :], (B, 16, D0))
    b1, r1 = _digit_step(rows0, ranks)
    eff1 = _first_occurrence(b1)
    lutA = jnp.full((B, D0), DEAD, jnp.int32).at[barr, b1].set(eff1)

    h1 = _make_pass(1)(xf, lutA.reshape(-1)).reshape(
        B, TPS, NSLOT, D1).sum(axis=1)
    rows1 = h1[barr, eff1]
    b2, r2 = _digit_step(rows1, r1)
    key2 = eff1 * D1 + b2
    eff2 = _first_occurrence(key2)
    lutB = jnp.full((B, NSLOT * D1), DEAD, jnp.int32).at[barr, key2].set(eff2)

    h2 = _make_pass(2)(xf, lutA.reshape(-1), lutB.reshape(-1)).reshape(
        B, TPS, NSLOT, D2).sum(axis=1)
    rows2 = h2[barr, eff2]
    b3, _ = _digit_step(rows2, r2)

    key32 = ((b1.astype(jnp.uint32) << 20) | (b2.astype(jnp.uint32) << 8)
             | b3.astype(jnp.uint32))
    pos_mask = key32 >> 31
    flip = jnp.where(pos_mask == 1, jnp.uint32(0x80000000),
                     jnp.uint32(0xFFFFFFFF))
    vals = lax.bitcast_convert_type(key32 ^ flip, jnp.float32)

    vlo = vals[:, jnp.asarray(LO_IDX)]
    vhi = vals[:, jnp.asarray(HI_IDX)]
    frac = jnp.asarray(FRAC)[None, :]
    return vlo * (jnp.float32(1) - frac) + vhi * frac


# speedup vs baseline: 83.4890x; 4.5937x over previous
import functools

import numpy as np
import jax
import jax.numpy as jnp
from jax import lax
from jax.experimental import pallas as pl
from jax.experimental.pallas import tpu as pltpu
from jax.experimental.pallas import tpu_sc as plsc

B = 8
N = 96 * 224 * 224
NC = 2
NS = 16
NW = NC * NS
TPS = NW // B
M = N // TPS
BLK = 6144
NBLK = M // BLK
NSLOT = 17
DEAD = 16
D0 = 4096
D1 = 4096
D2 = 256
UNROLL = 16

_QL = np.array([0.0, 0.01, 0.1, 0.25, 0.5, 0.75, 0.9, 0.99, 1.0], np.float32)
_pos = (_QL * np.float32(N - 1)).astype(np.float32)
_lo = np.floor(_pos).astype(np.int64)
_hi = np.ceil(_pos).astype(np.int64)
RANKS = sorted(set(_lo.tolist()) | set(_hi.tolist()))
while len(RANKS) < 16:
    RANKS.append(RANKS[-1])
assert len(RANKS) == 16, RANKS
_RIDX = {r: i for i, r in enumerate(RANKS)}
LO_IDX = np.array([_RIDX[int(r)] for r in _lo], np.int64)
HI_IDX = np.array([_RIDX[int(r)] for r in _hi], np.int64)
FRAC = (_pos - np.floor(_pos)).astype(np.float32)
RANKS_ARR = np.array(RANKS, np.int32)


@functools.lru_cache(maxsize=None)
def _make_pass(p):
    hsz = (D0, NSLOT * D1, NSLOT * D2)[p]
    lut_words = ((), (D0,), (D0, NSLOT * D1))[p]

    scratch = [pltpu.VMEM((BLK,), jnp.float32), pltpu.VMEM((BLK,), jnp.float32)]
    scratch += [pltpu.VMEM((w,), jnp.int32) for w in lut_words]
    scratch += [pltpu.VMEM((hsz,), jnp.int32),
                pltpu.SemaphoreType.DMA, pltpu.SemaphoreType.DMA]

    mesh = plsc.VectorSubcoreMesh(core_axis_name="c", subcore_axis_name="s")
    nlut = len(lut_words)

    @functools.partial(
        pl.kernel,
        out_type=jax.ShapeDtypeStruct((NW, hsz), jnp.int32),
        mesh=mesh,
        scratch_types=scratch,
        compiler_params=pltpu.CompilerParams(needs_layout_passes=False),
    )
    def body(x_hbm, *rest):
        lut_hbms = rest[:nlut]
        out_hbm = rest[nlut]
        buf0_v = rest[nlut + 1]
        buf1_v = rest[nlut + 2]
        lut_vs = rest[nlut + 3:nlut + 3 + nlut]
        hist_v, sem0, sem1 = rest[nlut + 3 + nlut:]

        cid = lax.axis_index("c")
        sid = lax.axis_index("s")
        wid = sid * NC + cid
        sample = wid // TPS
        base = wid * M

        pltpu.async_copy(x_hbm.at[pl.ds(base, BLK)], buf0_v, sem0)

        zero16 = jnp.zeros((16,), jnp.int32)

        @plsc.parallel_loop(0, hsz // 16, 1, unroll=UNROLL)
        def _zero(j):
            hist_v[pl.ds(j * 16, 16)] = zero16

        for lv, lh, w in zip(lut_vs, lut_hbms, lut_words):
            pltpu.sync_copy(lh.at[pl.ds(sample * w, w)], lv)

        one16 = jnp.ones((16,), jnp.int32)
        c31 = jnp.full((16,), 31, jnp.int32)
        cmin = jnp.full((16,), -2147483648, jnp.int32)

        def _digits(key):
            top = lax.shift_right_logical(key, jnp.full((16,), 20, jnp.int32))
            if p == 0:
                return (), top
            mid = lax.bitwise_and(
                lax.shift_right_logical(key, jnp.full((16,), 8, jnp.int32)),
                jnp.full((16,), 4095, jnp.int32))
            if p == 1:
                return (top,), mid
            low = lax.bitwise_and(key, jnp.full((16,), 255, jnp.int32))
            return (top, mid), low

        def _process(buf):
            @plsc.parallel_loop(0, BLK // 16, 1, unroll=UNROLL)
            def _vec(j):
                bits = plsc.bitcast(buf[pl.ds(j * 16, 16)], jnp.int32)
                m = lax.shift_right_arithmetic(bits, c31)
                key = lax.bitwise_xor(bits, lax.bitwise_or(m, cmin))
                prefix, digit = _digits(key)
                if p == 0:
                    idx = digit
                else:
                    slot = plsc.load_gather(lut_vs[0], [prefix[0]])
                    if p == 2:
                        slot = plsc.load_gather(
                            lut_vs[1], [slot * D1 + prefix[1]])
                    idx = slot * (D1 if p == 1 else D2) + digit
                plsc.addupdate_scatter(hist_v, [idx], one16)

        def _pair(q, c):
            i = q * 2
            pltpu.async_copy(x_hbm.at[pl.ds(base + (i + 1) * BLK, BLK)],
                             buf1_v, sem1)
            pltpu.make_async_copy(x_hbm.at[pl.ds(base + i * BLK, BLK)],
                                  buf0_v, sem0).wait()
            _process(buf0_v)
            nxt = jnp.where(i + 2 >= NBLK, 0, i + 2)
            pltpu.async_copy(x_hbm.at[pl.ds(base + nxt * BLK, BLK)],
                             buf0_v, sem0)
            pltpu.make_async_copy(x_hbm.at[pl.ds(base + (i + 1) * BLK, BLK)],
                                  buf1_v, sem1).wait()
            _process(buf1_v)
            return c

        lax.fori_loop(0, NBLK // 2, _pair, 0)

        pltpu.make_async_copy(x_hbm.at[pl.ds(base, BLK)], buf0_v,
                              sem0).wait()

        pltpu.sync_copy(hist_v, out_hbm.at[wid])

    return body


def _cumsum2(rows):
    k = rows.shape[-1]
    if k <= 256:
        return jnp.cumsum(rows, axis=-1)
    g = 128
    r = rows.reshape(rows.shape[:-1] + (k // g, g))
    inner = jnp.cumsum(r, axis=-1)
    block = jnp.cumsum(inner[..., -1], axis=-1) - inner[..., -1]
    return (inner + block[..., None]).reshape(rows.shape)


def _digit_step(rows, resid):
    cum = _cumsum2(rows)
    digit = jnp.sum((cum <= resid[..., None]).astype(jnp.int32), axis=-1)
    excl = cum - rows
    below = jnp.take_along_axis(excl, digit[..., None], axis=-1)[..., 0]
    return digit, resid - below


def _first_occurrence(keys):
    return jnp.sum((keys[:, None, :] < keys[:, :, None]).astype(jnp.int32),
                   axis=-1)


def kernel(x):
    xf = x.reshape(-1)
    barr = jnp.arange(B)[:, None]
    ranks = jnp.broadcast_to(jnp.asarray(RANKS_ARR)[None, :], (B, 16))

    h0 = _make_pass(0)(xf).reshape(B, TPS, D0).sum(axis=1)
    cum0 = _cumsum2(h0)
    b1 = jnp.sum((cum0[:, None, :] <= ranks[..., None]).astype(jnp.int32),
                 axis=-1)
    excl0 = cum0 - h0
    r1 = ranks - jnp.take_along_axis(excl0, b1, axis=-1)
    eff1 = _first_occurrence(b1)
    lutA = jnp.full((B, D0), DEAD, jnp.int32).at[barr, b1].set(eff1)

    h1 = _make_pass(1)(xf, lutA.reshape(-1)).reshape(
        B, TPS, NSLOT, D1).sum(axis=1)
    rows1 = h1[barr, eff1]
    b2, r2 = _digit_step(rows1, r1)
    key2 = eff1 * D1 + b2
    eff2 = _first_occurrence(key2)
    lutB = jnp.full((B, NSLOT * D1), DEAD, jnp.int32).at[barr, key2].set(eff2)

    h2 = _make_pass(2)(xf, lutA.reshape(-1), lutB.reshape(-1)).reshape(
        B, TPS, NSLOT, D2).sum(axis=1)
    rows2 = h2[barr, eff2]
    b3, _ = _digit_step(rows2, r2)

    key32 = ((b1.astype(jnp.uint32) << 20) | (b2.astype(jnp.uint32) << 8)
             | b3.astype(jnp.uint32))
    pos_mask = key32 >> 31
    flip = jnp.where(pos_mask == 1, jnp.uint32(0x80000000),
                     jnp.uint32(0xFFFFFFFF))
    vals = lax.bitcast_convert_type(key32 ^ flip, jnp.float32)

    vlo = vals[:, jnp.asarray(LO_IDX)]
    vhi = vals[:, jnp.asarray(HI_IDX)]
    frac = jnp.asarray(FRAC)[None, :]
    return vlo * (jnp.float32(1) - frac) + vhi * frac


# speedup vs baseline: 87.7952x; 1.0516x over previous
import functools

import numpy as np
import jax
import jax.numpy as jnp
from jax import lax
from jax.experimental import pallas as pl
from jax.experimental.pallas import tpu as pltpu
from jax.experimental.pallas import tpu_sc as plsc

B = 8
N = 96 * 224 * 224
NC = 2
NS = 16
NW = NC * NS
TPS = NW // B
M = N // TPS
BLK = 12288
NBLK = M // BLK
NSLOT = 17
DEAD = 16
D0 = 4096
D1 = 4096
D2 = 256
UNROLL = 16

_QL = np.array([0.0, 0.01, 0.1, 0.25, 0.5, 0.75, 0.9, 0.99, 1.0], np.float32)
_pos = (_QL * np.float32(N - 1)).astype(np.float32)
_lo = np.floor(_pos).astype(np.int64)
_hi = np.ceil(_pos).astype(np.int64)
RANKS = sorted(set(_lo.tolist()) | set(_hi.tolist()))
while len(RANKS) < 16:
    RANKS.append(RANKS[-1])
assert len(RANKS) == 16, RANKS
_RIDX = {r: i for i, r in enumerate(RANKS)}
LO_IDX = np.array([_RIDX[int(r)] for r in _lo], np.int64)
HI_IDX = np.array([_RIDX[int(r)] for r in _hi], np.int64)
FRAC = (_pos - np.floor(_pos)).astype(np.float32)
RANKS_ARR = np.array(RANKS, np.int32)


@functools.lru_cache(maxsize=None)
def _make_pass(p):
    hsz = (D0, NSLOT * D1, NSLOT * D2)[p]
    lut_words = ((), (D0,), (D0, NSLOT * D1))[p]

    scratch = [pltpu.VMEM((BLK,), jnp.float32), pltpu.VMEM((BLK,), jnp.float32)]
    scratch += [pltpu.VMEM((w,), jnp.int32) for w in lut_words]
    scratch += [pltpu.VMEM((hsz,), jnp.int32),
                pltpu.SemaphoreType.DMA, pltpu.SemaphoreType.DMA]

    mesh = plsc.VectorSubcoreMesh(core_axis_name="c", subcore_axis_name="s")
    nlut = len(lut_words)

    @functools.partial(
        pl.kernel,
        out_type=jax.ShapeDtypeStruct((NW, hsz), jnp.int32),
        mesh=mesh,
        scratch_types=scratch,
        compiler_params=pltpu.CompilerParams(needs_layout_passes=False),
    )
    def body(x_hbm, *rest):
        lut_hbms = rest[:nlut]
        out_hbm = rest[nlut]
        buf0_v = rest[nlut + 1]
        buf1_v = rest[nlut + 2]
        lut_vs = rest[nlut + 3:nlut + 3 + nlut]
        hist_v, sem0, sem1 = rest[nlut + 3 + nlut:]

        cid = lax.axis_index("c")
        sid = lax.axis_index("s")
        wid = sid * NC + cid
        sample = wid // TPS
        base = wid * M

        pltpu.async_copy(x_hbm.at[pl.ds(base, BLK)], buf0_v, sem0)

        zero16 = jnp.zeros((16,), jnp.int32)

        @plsc.parallel_loop(0, hsz // 16, 1, unroll=UNROLL)
        def _zero(j):
            hist_v[pl.ds(j * 16, 16)] = zero16

        for lv, lh, w in zip(lut_vs, lut_hbms, lut_words):
            pltpu.sync_copy(lh.at[pl.ds(sample * w, w)], lv)

        one16 = jnp.ones((16,), jnp.int32)
        c31 = jnp.full((16,), 31, jnp.int32)
        cmin = jnp.full((16,), -2147483648, jnp.int32)

        def _digits(key):
            top = lax.shift_right_logical(key, jnp.full((16,), 20, jnp.int32))
            if p == 0:
                return (), top
            mid = lax.bitwise_and(
                lax.shift_right_logical(key, jnp.full((16,), 8, jnp.int32)),
                jnp.full((16,), 4095, jnp.int32))
            if p == 1:
                return (top,), mid
            low = lax.bitwise_and(key, jnp.full((16,), 255, jnp.int32))
            return (top, mid), low

        def _process(buf):
            @plsc.parallel_loop(0, BLK // 16, 1, unroll=UNROLL)
            def _vec(j):
                bits = plsc.bitcast(buf[pl.ds(j * 16, 16)], jnp.int32)
                m = lax.shift_right_arithmetic(bits, c31)
                key = lax.bitwise_xor(bits, lax.bitwise_or(m, cmin))
                prefix, digit = _digits(key)
                if p == 0:
                    idx = digit
                else:
                    slot = plsc.load_gather(lut_vs[0], [prefix[0]])
                    if p == 2:
                        slot = plsc.load_gather(
                            lut_vs[1], [slot * D1 + prefix[1]])
                    idx = slot * (D1 if p == 1 else D2) + digit
                plsc.addupdate_scatter(hist_v, [idx], one16)

        def _pair(q, c):
            i = q * 2
            pltpu.async_copy(x_hbm.at[pl.ds(base + (i + 1) * BLK, BLK)],
                             buf1_v, sem1)
            pltpu.make_async_copy(x_hbm.at[pl.ds(base + i * BLK, BLK)],
                                  buf0_v, sem0).wait()
            _process(buf0_v)
            nxt = jnp.where(i + 2 >= NBLK, 0, i + 2)
            pltpu.async_copy(x_hbm.at[pl.ds(base + nxt * BLK, BLK)],
                             buf0_v, sem0)
            pltpu.make_async_copy(x_hbm.at[pl.ds(base + (i + 1) * BLK, BLK)],
                                  buf1_v, sem1).wait()
            _process(buf1_v)
            return c

        lax.fori_loop(0, NBLK // 2, _pair, 0)

        pltpu.make_async_copy(x_hbm.at[pl.ds(base, BLK)], buf0_v,
                              sem0).wait()

        pltpu.sync_copy(hist_v, out_hbm.at[wid])

    return body


def _cumsum2(rows):
    k = rows.shape[-1]
    if k <= 256:
        return jnp.cumsum(rows, axis=-1)
    g = 128
    r = rows.reshape(rows.shape[:-1] + (k // g, g))
    inner = jnp.cumsum(r, axis=-1)
    block = jnp.cumsum(inner[..., -1], axis=-1) - inner[..., -1]
    return (inner + block[..., None]).reshape(rows.shape)


def _digit_step(rows, resid):
    cum = _cumsum2(rows)
    digit = jnp.sum((cum <= resid[..., None]).astype(jnp.int32), axis=-1)
    excl = cum - rows
    below = jnp.take_along_axis(excl, digit[..., None], axis=-1)[..., 0]
    return digit, resid - below


def _first_occurrence(keys):
    return jnp.sum((keys[:, None, :] < keys[:, :, None]).astype(jnp.int32),
                   axis=-1)


def kernel(x):
    xf = x.reshape(-1)
    barr = jnp.arange(B)[:, None]
    ranks = jnp.broadcast_to(jnp.asarray(RANKS_ARR)[None, :], (B, 16))

    h0 = _make_pass(0)(xf).reshape(B, TPS, D0).sum(axis=1)
    cum0 = _cumsum2(h0)
    b1 = jnp.sum((cum0[:, None, :] <= ranks[..., None]).astype(jnp.int32),
                 axis=-1)
    excl0 = cum0 - h0
    r1 = ranks - jnp.take_along_axis(excl0, b1, axis=-1)
    eff1 = _first_occurrence(b1)
    lutA = jnp.full((B, D0), DEAD, jnp.int32).at[barr, b1].set(eff1)

    h1 = _make_pass(1)(xf, lutA.reshape(-1)).reshape(
        B, TPS, NSLOT, D1).sum(axis=1)
    rows1 = h1[barr, eff1]
    b2, r2 = _digit_step(rows1, r1)
    key2 = eff1 * D1 + b2
    eff2 = _first_occurrence(key2)
    lutB = jnp.full((B, NSLOT * D1), DEAD, jnp.int32).at[barr, key2].set(eff2)

    h2 = _make_pass(2)(xf, lutA.reshape(-1), lutB.reshape(-1)).reshape(
        B, TPS, NSLOT, D2).sum(axis=1)
    rows2 = h2[barr, eff2]
    b3, _ = _digit_step(rows2, r2)

    key32 = ((b1.astype(jnp.uint32) << 20) | (b2.astype(jnp.uint32) << 8)
             | b3.astype(jnp.uint32))
    pos_mask = key32 >> 31
    flip = jnp.where(pos_mask == 1, jnp.uint32(0x80000000),
                     jnp.uint32(0xFFFFFFFF))
    vals = lax.bitcast_convert_type(key32 ^ flip, jnp.float32)

    vlo = vals[:, jnp.asarray(LO_IDX)]
    vhi = vals[:, jnp.asarray(HI_IDX)]
    frac = jnp.asarray(FRAC)[None, :]
    return vlo * (jnp.float32(1) - frac) + vhi * frac
